# baseline (speedup 1.0000x reference)
import functools

import jax
import jax.numpy as jnp
from jax import lax
from jax.experimental import pallas as pl
from jax.experimental.pallas import tpu as pltpu
from jax.experimental.pallas import tpu_sc as plsc

_B, _F, _V = 16384, 26, 100000
_DE, _DH = 16, 32
_RE = _F * _DE
_RH = _F * _DH
_NW = 32
_EPW = _RE // _NW
_HPW = _RH // _NW
_CH = 4096
_NCH = _B // _CH


def _ser_body(xt_hbm, table_hbm, out_hbm, n_per_w, dlog,
              idx_v, buf_v, sem, sem_o):
    wid = lax.axis_index("s") * 2 + lax.axis_index("c")

    def do_row(r):
        f = r // dlog
        pltpu.sync_copy(xt_hbm.at[f], idx_v)

        outs = [None, None]
        for c in range(_NCH):
            b = c % 2
            sl = pl.ds(c * _CH, _CH)
            if outs[b] is not None:
                outs[b].wait()
            pltpu.async_copy(table_hbm.at[r].at[idx_v.at[sl]],
                             buf_v.at[b], sem).wait()
            outs[b] = pltpu.async_copy(buf_v.at[b], out_hbm.at[r, sl], sem_o)
        for cp in outs:
            cp.wait()

    def row(t, carry):
        do_row(wid * n_per_w + t)
        return carry

    lax.fori_loop(0, n_per_w, row, 0)


@functools.partial(jax.jit, static_argnums=(2, 3, 4))
def _ser_one(xt, table_t, rows, n_per_w, dlog):
    mesh = plsc.VectorSubcoreMesh(core_axis_name="c", subcore_axis_name="s")

    def wrapped(xt_hbm, table_hbm, out_hbm, idx_v, buf_v, sem, sem_o):
        _ser_body(xt_hbm, table_hbm, out_hbm, n_per_w, dlog,
                  idx_v, buf_v, sem, sem_o)

    return pl.kernel(
        wrapped,
        out_type=jax.ShapeDtypeStruct((rows, _B), jnp.float32),
        mesh=mesh,
        scratch_types=[
            pltpu.VMEM((_B,), jnp.int32),
            pltpu.VMEM((2, _CH), jnp.float32),
            pltpu.SemaphoreType.DMA,
            pltpu.SemaphoreType.DMA,
        ],
        compiler_params=pltpu.CompilerParams(use_tc_tiling_on_sc=False),
    )(xt, table_t)


def kernel(X, easy_table, hard_table):
    xt = X.T
    easy_t = jnp.transpose(easy_table, (0, 2, 1)).reshape(_RE, _V)
    hard_t = jnp.transpose(hard_table, (0, 2, 1)).reshape(_RH, _V)
    oute_t = _ser_one(xt, easy_t, _RE, _EPW, _DE)
    outh_t = _ser_one(xt, hard_t, _RH, _HPW, _DH)
    return (oute_t.T.reshape(_B, _RE), outh_t.T.reshape(_B, _RH))
